# baseline (speedup 1.0000x reference)
import jax
import jax.numpy as jnp
from jax import lax
from jax.experimental import pallas as pl
from jax.experimental.pallas import tpu as pltpu
from jax.experimental.pallas import tpu_sc as plsc

N_LATENT = 32
BATCH = 16384

NC = 2
NS = 16
L = 16
NW = NC * NS
B_PER_W = BATCH // NW
MB = 128
NMB = B_PER_W // MB
NG = MB // L
DEPTH = 6


def _mf_body(uids_hbm, iids_hbm, ubias_hbm, ibias_hbm, ulat_hbm, ilat_hbm,
             out_hbm,
             uidx_v, iidx_v, ubias_v, ibias_v, out_v,
             ubuf, ibuf, sems, sem_bias):
    wid = lax.axis_index("s")
    base = wid * B_PER_W

    pltpu.sync_copy(uids_hbm.at[pl.ds(base, B_PER_W)], uidx_v)
    pltpu.sync_copy(iids_hbm.at[pl.ds(base, B_PER_W)], iidx_v)

    c_ub = pltpu.async_copy(ubias_hbm.at[uidx_v], ubias_v, sem_bias)
    c_ib = pltpu.async_copy(ibias_hbm.at[iidx_v], ibias_v, sem_bias)

    lanes0 = lax.iota(jnp.int32, L)

    def fire_group(g, p, sem):
        ubase = (uidx_v[pl.ds(g * L, L)] // L) * L
        ibase = (iidx_v[pl.ds(g * L, L)] // L) * L
        for r in range(L):
            cu = pl.multiple_of(ubase[r], L)
            ci = pl.multiple_of(ibase[r], L)
            pltpu.async_copy(ulat_hbm.at[:, :, pl.ds(cu, L)],
                             ubuf.at[p, :, :, pl.ds(r * L, L)], sem)
            pltpu.async_copy(ilat_hbm.at[:, :, pl.ds(ci, L)],
                             ibuf.at[p, :, :, pl.ds(r * L, L)], sem)

    def drain_compute(g, p, sem):
        pltpu.make_async_copy(ulat_hbm.at[:, :, pl.ds(0, L * L)],
                              ubuf.at[p], sem).wait()
        pltpu.make_async_copy(ilat_hbm.at[:, :, pl.ds(0, L * L)],
                              ibuf.at[p], sem).wait()
        s0 = g * L
        pv = jnp.full((L,), p, jnp.int32)
        ulane = lanes0 * L + uidx_v[pl.ds(s0, L)] % L
        ilane = lanes0 * L + iidx_v[pl.ds(s0, L)] % L
        acc = jnp.zeros((L,), jnp.float32)
        for c in range(4):
            cv = jnp.full((L,), c, jnp.int32)
            for s in range(8):
                sv = jnp.full((L,), s, jnp.int32)
                u = plsc.load_gather(ubuf, [pv, cv, sv, ulane])
                v = plsc.load_gather(ibuf, [pv, cv, sv, ilane])
                acc = acc + u * v
        out_v[pl.ds(s0, L)] = acc

    NGRP = B_PER_W // L

    def body(g, _):
        @pl.when(g < NGRP)
        def _():
            fire_group(g, g % DEPTH, sems.at[g % DEPTH])

        @pl.when(g >= DEPTH - 1)
        def _():
            gm = g - (DEPTH - 1)
            drain_compute(gm, gm % DEPTH, sems.at[gm % DEPTH])

        return 0

    lax.fori_loop(0, NGRP + DEPTH - 1, body, 0)

    c_ub.wait()
    c_ib.wait()

    def finish_body(g, _):
        s0 = g * L
        x = out_v[pl.ds(s0, L)] + ubias_v[pl.ds(s0, L)] + ibias_v[pl.ds(s0, L)]
        out_v[pl.ds(s0, L)] = 1.0 / (1.0 + jnp.exp(-x))
        return 0

    lax.fori_loop(0, NGRP, finish_body, 0)

    pltpu.sync_copy(out_v, out_hbm.at[pl.ds(base, B_PER_W)])


@jax.jit
def _mf(user_ids, item_ids, user_bias_w, item_bias_w, user_latent_w,
        item_latent_w):
    mesh = plsc.VectorSubcoreMesh(core_axis_name="c", subcore_axis_name="s",
                                  num_cores=1, num_subcores=NS)
    run = pl.kernel(
        _mf_body,
        out_type=jax.ShapeDtypeStruct((BATCH // 2,), jnp.float32),
        mesh=mesh,
        scratch_types=[
            pltpu.VMEM((B_PER_W,), jnp.int32),
            pltpu.VMEM((B_PER_W,), jnp.int32),
            pltpu.VMEM((B_PER_W,), jnp.float32),
            pltpu.VMEM((B_PER_W,), jnp.float32),
            pltpu.VMEM((B_PER_W,), jnp.float32),
            pltpu.VMEM((DEPTH, 4, 8, L * L), jnp.float32),
            pltpu.VMEM((DEPTH, 4, 8, L * L), jnp.float32),
            pltpu.SemaphoreType.DMA((DEPTH,)),
            pltpu.SemaphoreType.DMA,
        ],
        compiler_params=pltpu.CompilerParams(needs_layout_passes=False,
                                             use_tc_tiling_on_sc=True),
    )
    ulat3 = user_latent_w.T.reshape(4, 8, -1)
    ilat3 = item_latent_w.T.reshape(4, 8, -1)
    ub = user_bias_w.reshape(-1)
    ib = item_bias_w.reshape(-1)
    h = BATCH // 2
    out0 = run(user_ids[:h], item_ids[:h], ub, ib, ulat3, ilat3)
    out1 = run(user_ids[h:], item_ids[h:], ub, ib, ulat3, ilat3)
    return jnp.concatenate([out0, out1])


def kernel(user_ids, item_ids, user_bias_w, item_bias_w, user_latent_w,
           item_latent_w):
    return _mf(user_ids.astype(jnp.int32), item_ids.astype(jnp.int32),
               user_bias_w, item_bias_w, user_latent_w, item_latent_w)


# speedup vs baseline: 1.5219x; 1.5219x over previous
import jax
import jax.numpy as jnp
from jax import lax
from jax.experimental import pallas as pl
from jax.experimental.pallas import tpu as pltpu
from jax.experimental.pallas import tpu_sc as plsc

N_LATENT = 32
BATCH = 16384

NC = 2
NS = 16
L = 16
NW = NC * NS
B_PER_W = BATCH // NW
MB = 128
NMB = B_PER_W // MB
NG = MB // L
DEPTH = 6


def _mf_body(uids_hbm, iids_hbm, ubias_hbm, ibias_hbm, ulat_hbm, ilat_hbm,
             out_hbm,
             uidx_v, iidx_v, ubias_v, ibias_v, out_v,
             ubuf, ibuf, sems, sem_bias):
    wid = lax.axis_index("s") * NC + lax.axis_index("c")
    base = wid * B_PER_W

    pltpu.sync_copy(uids_hbm.at[pl.ds(base, B_PER_W)], uidx_v)
    pltpu.sync_copy(iids_hbm.at[pl.ds(base, B_PER_W)], iidx_v)

    c_ub = pltpu.async_copy(ubias_hbm.at[uidx_v], ubias_v, sem_bias)
    c_ib = pltpu.async_copy(ibias_hbm.at[iidx_v], ibias_v, sem_bias)

    lanes0 = lax.iota(jnp.int32, L)

    def fire_group(g, p, sem):
        ubase = (uidx_v[pl.ds(g * L, L)] // L) * L
        ibase = (iidx_v[pl.ds(g * L, L)] // L) * L
        for r in range(L):
            cu = pl.multiple_of(ubase[r], L)
            ci = pl.multiple_of(ibase[r], L)
            pltpu.async_copy(ulat_hbm.at[:, :, pl.ds(cu, L)],
                             ubuf.at[p, :, :, pl.ds(r * L, L)], sem)
            pltpu.async_copy(ilat_hbm.at[:, :, pl.ds(ci, L)],
                             ibuf.at[p, :, :, pl.ds(r * L, L)], sem)

    def drain_compute(g, p, sem):
        pltpu.make_async_copy(ulat_hbm.at[:, :, pl.ds(0, L * L)],
                              ubuf.at[p], sem).wait()
        pltpu.make_async_copy(ilat_hbm.at[:, :, pl.ds(0, L * L)],
                              ibuf.at[p], sem).wait()
        s0 = g * L
        pv = jnp.full((L,), p, jnp.int32)
        ulane = lanes0 * L + uidx_v[pl.ds(s0, L)] % L
        ilane = lanes0 * L + iidx_v[pl.ds(s0, L)] % L
        acc = jnp.zeros((L,), jnp.float32)
        for c in range(4):
            cv = jnp.full((L,), c, jnp.int32)
            for s in range(8):
                sv = jnp.full((L,), s, jnp.int32)
                u = plsc.load_gather(ubuf, [pv, cv, sv, ulane])
                v = plsc.load_gather(ibuf, [pv, cv, sv, ilane])
                acc = acc + u * v
        out_v[pl.ds(s0, L)] = acc

    NGRP = B_PER_W // L

    def body(g, _):
        @pl.when(g < NGRP)
        def _():
            fire_group(g, g % DEPTH, sems.at[g % DEPTH])

        @pl.when(g >= DEPTH - 1)
        def _():
            gm = g - (DEPTH - 1)
            drain_compute(gm, gm % DEPTH, sems.at[gm % DEPTH])

        return 0

    lax.fori_loop(0, NGRP + DEPTH - 1, body, 0)

    c_ub.wait()
    c_ib.wait()

    def finish_body(g, _):
        s0 = g * L
        x = out_v[pl.ds(s0, L)] + ubias_v[pl.ds(s0, L)] + ibias_v[pl.ds(s0, L)]
        out_v[pl.ds(s0, L)] = 1.0 / (1.0 + jnp.exp(-x))
        return 0

    lax.fori_loop(0, NGRP, finish_body, 0)

    pltpu.sync_copy(out_v, out_hbm.at[pl.ds(base, B_PER_W)])


@jax.jit
def _mf(user_ids, item_ids, user_bias_w, item_bias_w, user_latent_w,
        item_latent_w):
    mesh = plsc.VectorSubcoreMesh(core_axis_name="c", subcore_axis_name="s",
                                  num_cores=NC, num_subcores=NS)
    run = pl.kernel(
        _mf_body,
        out_type=jax.ShapeDtypeStruct((BATCH,), jnp.float32),
        mesh=mesh,
        scratch_types=[
            pltpu.VMEM((B_PER_W,), jnp.int32),
            pltpu.VMEM((B_PER_W,), jnp.int32),
            pltpu.VMEM((B_PER_W,), jnp.float32),
            pltpu.VMEM((B_PER_W,), jnp.float32),
            pltpu.VMEM((B_PER_W,), jnp.float32),
            pltpu.VMEM((DEPTH, 4, 8, L * L), jnp.float32),
            pltpu.VMEM((DEPTH, 4, 8, L * L), jnp.float32),
            pltpu.SemaphoreType.DMA((DEPTH,)),
            pltpu.SemaphoreType.DMA,
        ],
        compiler_params=pltpu.CompilerParams(needs_layout_passes=False,
                                             use_tc_tiling_on_sc=True),
    )
    ulat3 = user_latent_w.T.reshape(4, 8, -1)
    ilat3 = item_latent_w.T.reshape(4, 8, -1)
    return run(user_ids, item_ids,
               user_bias_w.reshape(-1), item_bias_w.reshape(-1),
               ulat3, ilat3)


def kernel(user_ids, item_ids, user_bias_w, item_bias_w, user_latent_w,
           item_latent_w):
    return _mf(user_ids.astype(jnp.int32), item_ids.astype(jnp.int32),
               user_bias_w, item_bias_w, user_latent_w, item_latent_w)
